# Initial kernel scaffold; baseline (speedup 1.0000x reference)
import jax
import jax.numpy as jnp
from jax.experimental import pallas as pl


def kernel(x, edge_index, W, b, Wc, bc):
    raise NotImplementedError("write your pallas kernel here")


# speedup vs baseline: 9.6038x; 9.6038x over previous
import functools

import jax
import jax.numpy as jnp
from jax import lax
from jax.experimental import pallas as pl
from jax.experimental.pallas import tpu as pltpu
from jax.experimental.pallas import tpu_sc as plsc

NC = 2
NS = 16
NW = NC * NS
K = 80


def _mesh():
    return plsc.VectorSubcoreMesh(core_axis_name="c", subcore_axis_name="s")


def _deg_call(dst3, ones_hbm, zeros_hbm, n_pad, e):
    ept = e // NW
    nchunks = ept // K
    rows_pt = n_pad // NS

    @functools.partial(
        pl.kernel,
        mesh=_mesh(),
        out_type=jax.ShapeDtypeStruct((NC * n_pad,), jnp.float32),
        scratch_types=[
            pltpu.VMEM((nchunks, K), jnp.int32),
            pltpu.VMEM((K,), jnp.float32),
            pltpu.VMEM_SHARED((n_pad,), jnp.float32),
        ],
    )
    def kfn(dst_hbm, ones_h, zeros_h, out_hbm, di, ones_v, dacc):
        c = lax.axis_index("c")
        s = lax.axis_index("s")
        w = c * NS + s
        pltpu.sync_copy(ones_h, ones_v)
        pltpu.sync_copy(dst_hbm.at[w], di)
        pltpu.sync_copy(zeros_h, dacc.at[pl.ds(s * rows_pt, rows_pt)])
        plsc.subcore_barrier()

        def body(j, carry):
            pltpu.sync_copy(ones_v, dacc.at[di.at[j]], add=True)
            return carry

        lax.fori_loop(0, nchunks, body, 0)
        plsc.subcore_barrier()
        pltpu.sync_copy(
            dacc.at[pl.ds(s * rows_pt, rows_pt)],
            out_hbm.at[pl.ds(c * n_pad + s * rows_pt, rows_pt)],
        )

    return kfn(dst3, ones_hbm, zeros_hbm)


def _scatter_call(u, src3, dst3, zrows_hbm, n_pad, e):
    ept = e // NW
    nchunks = ept // K
    rows_pt = n_pad // NS
    d = u.shape[1]

    @functools.partial(
        pl.kernel,
        mesh=_mesh(),
        out_type=jax.ShapeDtypeStruct((NC, n_pad, d), jnp.float32),
        scratch_types=[
            pltpu.VMEM((nchunks, K), jnp.int32),
            pltpu.VMEM((nchunks, K), jnp.int32),
            pltpu.VMEM((K, d), jnp.float32),
            pltpu.VMEM((16, d), jnp.float32),
            pltpu.VMEM_SHARED((n_pad, d), jnp.float32),
            pltpu.SemaphoreType.DMA,
        ],
    )
    def kfn(u_hbm, src_hbm, dst_hbm, zr_hbm, out_hbm, si, di, rows, zb, acc, sem):
        c = lax.axis_index("c")
        s = lax.axis_index("s")
        w = c * NS + s
        pltpu.sync_copy(src_hbm.at[w], si)
        pltpu.sync_copy(dst_hbm.at[w], di)
        pltpu.sync_copy(zr_hbm, zb)

        def zbody(t, carry):
            pltpu.sync_copy(zb, acc.at[pl.ds(s * rows_pt + t * 16, 16)])
            return carry

        lax.fori_loop(0, rows_pt // 16, zbody, 0)
        plsc.subcore_barrier()

        def body(j, carry):
            pltpu.async_copy(u_hbm.at[si.at[j]], rows, sem).wait()
            pltpu.sync_copy(rows, acc.at[di.at[j]], add=True)
            return carry

        lax.fori_loop(0, nchunks, body, 0)
        plsc.subcore_barrier()

        def wbody(t, carry):
            r0 = s * rows_pt + t * K
            pltpu.sync_copy(acc.at[pl.ds(r0, K)], out_hbm.at[c, pl.ds(r0, K)])
            return carry

        lax.fori_loop(0, rows_pt // K, wbody, 0)

    return kfn(u, src3, dst3, zrows_hbm)


def _edge_call(tcat_flat, sidx3, didx3, stage_idx, te_pad):
    vpt = 2 * (te_pad // NW)
    ck = 128
    nchunks = vpt // ck

    @functools.partial(
        pl.kernel,
        mesh=_mesh(),
        out_type=jax.ShapeDtypeStruct((2 * te_pad,), jnp.float32),
        scratch_types=[
            pltpu.VMEM((nchunks, ck), jnp.int32),
            pltpu.VMEM((nchunks, ck), jnp.int32),
            pltpu.VMEM((ck,), jnp.int32),
            pltpu.VMEM((ck,), jnp.float32),
            pltpu.VMEM((ck,), jnp.float32),
            pltpu.VMEM_SHARED((NS * ck,), jnp.float32),
            pltpu.SemaphoreType.DMA,
            pltpu.SemaphoreType.DMA,
        ],
    )
    def kfn(tab_hbm, src_hbm, dst_hbm, sidx_hbm, out_hbm,
            si, di, aidx, ra, rb, stage, sem_a, sem_b):
        c = lax.axis_index("c")
        s = lax.axis_index("s")
        w = c * NS + s
        pltpu.sync_copy(src_hbm.at[w], si)
        pltpu.sync_copy(dst_hbm.at[w], di)
        pltpu.sync_copy(sidx_hbm.at[s], aidx)
        base = w * vpt

        def body(j, carry):
            cpa = pltpu.async_copy(tab_hbm.at[si.at[j]], ra, sem_a)
            cpb = pltpu.async_copy(tab_hbm.at[di.at[j]], rb, sem_b)
            cpa.wait()
            cpb.wait()
            pltpu.sync_copy(ra, stage.at[pl.ds(s * ck, ck)])
            pltpu.sync_copy(rb, stage.at[aidx], add=True)
            pltpu.sync_copy(stage.at[pl.ds(s * ck, ck)],
                            out_hbm.at[pl.ds(base + j * ck, ck)])
            return carry

        lax.fori_loop(0, nchunks, body, 0)

    return kfn(tcat_flat, sidx3, didx3, stage_idx)


def _scale_matmul_call(x_pad, w, dega, degb):
    def body(x_ref, w_ref, da_ref, db_ref, u_ref):
        dinv = lax.rsqrt(da_ref[...] + db_ref[...] + 1.0)
        xw = jnp.dot(x_ref[...], w_ref[...], preferred_element_type=jnp.float32)
        u_ref[...] = xw * dinv

    return pl.pallas_call(
        body, out_shape=jax.ShapeDtypeStruct(x_pad.shape, jnp.float32)
    )(x_pad, w, dega, degb)


def _final_tc_call(s0, s1, u, dega, degb, b2, wcat, bc4):
    n_pad, d = u.shape

    def body(s0_ref, s1_ref, u_ref, da_ref, db_ref, b_ref, wc_ref, bc_ref,
             o_ref):
        dinv = lax.rsqrt(da_ref[...] + db_ref[...] + 1.0)
        h = jnp.maximum(
            dinv * (s0_ref[...] + s1_ref[...] + u_ref[...]) + b_ref[...], 0.0
        )
        o_ref[...] = (
            jnp.dot(h, wc_ref[...], preferred_element_type=jnp.float32)
            + bc_ref[...]
        )

    return pl.pallas_call(
        body, out_shape=jax.ShapeDtypeStruct((n_pad, 4), jnp.float32)
    )(s0, s1, u, dega, degb, b2, wcat, bc4)


def kernel(x, edge_index, W, b, Wc, bc):
    n, d_in = x.shape
    d_out = W.shape[1]
    e = edge_index.shape[1]
    n_pad = ((n + 16 * NS - 1) // (16 * NS)) * (16 * NS)
    ept = e // NW

    src = edge_index[0]
    dst = edge_index[1]

    neg_key = jax.random.key(42)
    ka, kb = jax.random.split(neg_key)
    neg_src = jax.random.randint(ka, (e,), 0, n, dtype=edge_index.dtype)
    neg_dst = jax.random.randint(kb, (e,), 0, n, dtype=edge_index.dtype)
    all_src = jnp.concatenate([src, neg_src])
    all_dst = jnp.concatenate([dst, neg_dst])

    src3 = src.reshape(NW, ept // K, K)
    dst3 = dst.reshape(NW, ept // K, K)

    te = 2 * e
    ck = 128
    te_pad = ((te + NW * (ck // 2) - 1) // (NW * (ck // 2))) * (NW * (ck // 2))
    asrc = jnp.pad(all_src, (0, te_pad - te))
    adst = jnp.pad(all_dst, (0, te_pad - te))
    two = jnp.arange(2, dtype=jnp.int32)
    sidx3 = (4 * asrc[:, None] + two).reshape(NW, -1, ck)
    didx3 = (4 * adst[:, None] + (two + 2)).reshape(NW, -1, ck)

    ones_k = jnp.ones((K,), jnp.float32)
    zeros_row = jnp.zeros((n_pad // NS,), jnp.float32)
    zeros_16d = jnp.zeros((16, d_out), jnp.float32)

    deg2 = _deg_call(dst3, ones_k, zeros_row, n_pad, e)
    dega = deg2[:n_pad].reshape(n_pad, 1)
    degb = deg2[n_pad:].reshape(n_pad, 1)

    x_pad = jnp.pad(x, ((0, n_pad - n), (0, 0)))
    u = _scale_matmul_call(x_pad, W, dega, degb)

    s2 = _scatter_call(u, src3, dst3, zeros_16d, n_pad, e)

    wcat = jnp.concatenate([Wc[:, :d_out].T, Wc[:, d_out:].T], axis=1)
    bc4 = jnp.concatenate([bc, jnp.zeros((2,), jnp.float32)]).reshape(1, 4)
    proj = _final_tc_call(
        s2[0], s2[1], u, dega, degb, b.reshape(1, d_out), wcat, bc4
    )

    stage_idx = (jnp.arange(NS, dtype=jnp.int32)[:, None] * ck
                 + jnp.arange(ck, dtype=jnp.int32)[None, :])
    out_flat = _edge_call(
        proj.reshape(n_pad * 4), sidx3, didx3, stage_idx, te_pad
    )
    return out_flat[: 2 * te].reshape(te, 2)
